# baseline (device time: 4252429 ns/iter reference)
import jax
import jax.numpy as jnp
from jax import lax
from jax.experimental import pallas as pl
from jax.experimental.pallas import tpu as pltpu

N_CHUNKS = 16


def kernel(x):
    m, n = x.shape
    half = m // 2
    ch = half // N_CHUNKS

    def body(x_ref, out_ref, local_sems, x_send, x_recv, y_send, y_recv):
        my_x = lax.axis_index("x")
        my_y = lax.axis_index("y")
        other_x = 1 - my_x

        lch = m // N_CHUNKS
        locals_ = []
        for c in range(N_CHUNKS):
            cp = pltpu.make_async_copy(
                x_ref.at[pl.ds(c * lch, lch), :],
                out_ref.at[pl.ds(my_x * m + c * lch, lch), :],
                local_sems.at[c],
            )
            cp.start()
            locals_.append(cp)

        x_rdmas = []
        for c in range(N_CHUNKS):
            row = my_y * half + c * ch
            r = pltpu.make_async_remote_copy(
                src_ref=x_ref.at[pl.ds(row, ch), :],
                dst_ref=out_ref.at[pl.ds(my_x * m + row, ch), :],
                send_sem=x_send.at[c],
                recv_sem=x_recv.at[c],
                device_id=(other_x, my_y),
                device_id_type=pl.DeviceIdType.MESH,
            )
            r.start()
            x_rdmas.append(r)

        y_rdmas = []
        for c in range(N_CHUNKS):
            x_rdmas[c].wait_recv()
            row = other_x * m + my_y * half + c * ch
            f = pltpu.make_async_remote_copy(
                src_ref=out_ref.at[pl.ds(row, ch), :],
                dst_ref=out_ref.at[pl.ds(row, ch), :],
                send_sem=y_send.at[c],
                recv_sem=y_recv.at[c],
                device_id=(my_x, 1 - my_y),
                device_id_type=pl.DeviceIdType.MESH,
            )
            f.start()
            y_rdmas.append(f)

        for c in range(N_CHUNKS):
            y_rdmas[c].wait_recv()
        for c in range(N_CHUNKS):
            x_rdmas[c].wait_send()
            y_rdmas[c].wait_send()
            locals_[c].wait()

    return pl.pallas_call(
        body,
        out_shape=jax.ShapeDtypeStruct((2 * m, n), x.dtype),
        in_specs=[pl.BlockSpec(memory_space=pl.ANY)],
        out_specs=pl.BlockSpec(memory_space=pl.ANY),
        scratch_shapes=[
            pltpu.SemaphoreType.DMA((N_CHUNKS,)),
            pltpu.SemaphoreType.DMA((N_CHUNKS,)),
            pltpu.SemaphoreType.DMA((N_CHUNKS,)),
            pltpu.SemaphoreType.DMA((N_CHUNKS,)),
            pltpu.SemaphoreType.DMA((N_CHUNKS,)),
        ],
    )(x)


# device time: 945440 ns/iter; 4.4978x vs baseline; 4.4978x over previous
import jax
import jax.numpy as jnp
from jax import lax
from jax.experimental import pallas as pl
from jax.experimental.pallas import tpu as pltpu

N_CHUNKS = 16
STAGE_ROWS = 2048


def kernel(x):
    m, n = x.shape
    half = m // 2
    ch = half // N_CHUNKS
    n_stage = m // STAGE_ROWS

    def body(x_ref, out_ref, stage_vmem, ld_sems, st_sems,
             x_send, x_recv, y_send, y_recv):
        my_x = lax.axis_index("x")
        my_y = lax.axis_index("y")
        other_x = 1 - my_x
        base = my_x * m

        x_rdmas = []
        for c in range(N_CHUNKS):
            row = my_y * half + c * ch
            r = pltpu.make_async_remote_copy(
                src_ref=x_ref.at[pl.ds(row, ch), :],
                dst_ref=out_ref.at[pl.ds(base + row, ch), :],
                send_sem=x_send.at[c],
                recv_sem=x_recv.at[c],
                device_id=(other_x, my_y),
                device_id_type=pl.DeviceIdType.MESH,
            )
            r.start()
            x_rdmas.append(r)

        stage_stores = [None, None]
        next_stage = [0]

        def do_stage_chunk():
            c = next_stage[0]
            if c >= n_stage:
                return
            next_stage[0] = c + 1
            slot = c % 2
            if stage_stores[slot] is not None:
                stage_stores[slot].wait()
            ld = pltpu.make_async_copy(
                x_ref.at[pl.ds(c * STAGE_ROWS, STAGE_ROWS), :],
                stage_vmem.at[slot],
                ld_sems.at[slot],
            )
            ld.start()
            ld.wait()
            st = pltpu.make_async_copy(
                stage_vmem.at[slot],
                out_ref.at[pl.ds(base + c * STAGE_ROWS, STAGE_ROWS), :],
                st_sems.at[slot],
            )
            st.start()
            stage_stores[slot] = st

        y_rdmas = []
        for c in range(N_CHUNKS):
            x_rdmas[c].wait_recv()
            row = other_x * m + my_y * half + c * ch
            f = pltpu.make_async_remote_copy(
                src_ref=out_ref.at[pl.ds(row, ch), :],
                dst_ref=out_ref.at[pl.ds(row, ch), :],
                send_sem=y_send.at[c],
                recv_sem=y_recv.at[c],
                device_id=(my_x, 1 - my_y),
                device_id_type=pl.DeviceIdType.MESH,
            )
            f.start()
            y_rdmas.append(f)
            do_stage_chunk()

        while next_stage[0] < n_stage:
            do_stage_chunk()

        for c in range(N_CHUNKS):
            y_rdmas[c].wait_recv()
        for c in range(N_CHUNKS):
            x_rdmas[c].wait_send()
            y_rdmas[c].wait_send()
        stage_stores[0].wait()
        stage_stores[1].wait()

    return pl.pallas_call(
        body,
        out_shape=jax.ShapeDtypeStruct((2 * m, n), x.dtype),
        in_specs=[pl.BlockSpec(memory_space=pl.ANY)],
        out_specs=pl.BlockSpec(memory_space=pl.ANY),
        scratch_shapes=[
            pltpu.VMEM((2, STAGE_ROWS, n), x.dtype),
            pltpu.SemaphoreType.DMA((2,)),
            pltpu.SemaphoreType.DMA((2,)),
            pltpu.SemaphoreType.DMA((N_CHUNKS,)),
            pltpu.SemaphoreType.DMA((N_CHUNKS,)),
            pltpu.SemaphoreType.DMA((N_CHUNKS,)),
            pltpu.SemaphoreType.DMA((N_CHUNKS,)),
        ],
    )(x)
